# Initial kernel scaffold; baseline (speedup 1.0000x reference)
import jax
import jax.numpy as jnp
from jax.experimental import pallas as pl


def kernel(x, edge_index, W0, b0, W1, b1, W2, b2, W3, b3, g0, be0, g1, be1, g2, be2):
    raise NotImplementedError("write your pallas kernel here")


# speedup vs baseline: 11.4994x; 11.4994x over previous
import functools

import jax
import jax.numpy as jnp
from jax import lax
from jax.experimental import pallas as pl
from jax.experimental.pallas import tpu as pltpu
from jax.experimental.pallas import tpu_sc as plsc

N = 10000
E = 320000
D = 128
EPS = 1e-5

NC = 2
NS = 16
NW = NC * NS
E_PER_TILE = E // NW
CHUNK = 80
NCHUNKS = E_PER_TILE // CHUNK
N_PAD = 10240
RPT = N_PAD // NS

_MESH = plsc.VectorSubcoreMesh(core_axis_name="c", subcore_axis_name="s")


def _sc_scatter_wide(h, src, dst, zblk):
    @functools.partial(
        pl.kernel,
        out_type=jax.ShapeDtypeStruct((NC, N_PAD, D), jnp.float32),
        mesh=_MESH,
        scratch_types=[
            pltpu.VMEM((CHUNK,), jnp.int32),
            pltpu.VMEM((CHUNK,), jnp.int32),
            pltpu.VMEM((CHUNK, D), jnp.float32),
            pltpu.VMEM_SHARED((N_PAD, D), jnp.float32),
            pltpu.SemaphoreType.DMA,
        ],
    )
    def k(h_hbm, src_hbm, dst_hbm, z_hbm, out_hbm, src_v, dst_v, rows_v,
          acc_sp, sem):
        cid = lax.axis_index("c")
        sid = lax.axis_index("s")
        r0 = sid * RPT
        pltpu.sync_copy(z_hbm, acc_sp.at[pl.ds(r0, RPT)])
        plsc.subcore_barrier()

        ebase = (cid * NS + sid) * E_PER_TILE

        def body(i, carry):
            base = pl.multiple_of(ebase + i * CHUNK, 8)
            pltpu.sync_copy(src_hbm.at[pl.ds(base, CHUNK)], src_v)
            pltpu.sync_copy(dst_hbm.at[pl.ds(base, CHUNK)], dst_v)
            pltpu.async_copy(h_hbm.at[src_v], rows_v, sem).wait()
            pltpu.sync_copy(rows_v, acc_sp.at[dst_v], add=True)
            return carry

        lax.fori_loop(0, NCHUNKS, body, 0)
        plsc.subcore_barrier()
        pltpu.sync_copy(acc_sp.at[pl.ds(r0, RPT)],
                        out_hbm.at[cid, pl.ds(r0, RPT)])

    return k(h, src, dst, zblk)


def _sc_deg(dst):
    @functools.partial(
        pl.kernel,
        out_type=jax.ShapeDtypeStruct((NC, N_PAD), jnp.float32),
        mesh=_MESH,
        scratch_types=[
            pltpu.VMEM((CHUNK,), jnp.int32),
            pltpu.VMEM((CHUNK,), jnp.float32),
            pltpu.VMEM((RPT,), jnp.float32),
            pltpu.VMEM_SHARED((N_PAD,), jnp.float32),
        ],
    )
    def k(dst_hbm, out_hbm, dst_v, ones_v, zbuf, acc_sp):
        cid = lax.axis_index("c")
        sid = lax.axis_index("s")
        for j in range(CHUNK // 16):
            ones_v[pl.ds(j * 16, 16)] = jnp.ones((16,), jnp.float32)
        for j in range(RPT // 16):
            zbuf[pl.ds(j * 16, 16)] = jnp.zeros((16,), jnp.float32)
        r0 = sid * RPT
        pltpu.sync_copy(zbuf, acc_sp.at[pl.ds(r0, RPT)])
        plsc.subcore_barrier()

        ebase = (cid * NS + sid) * E_PER_TILE

        def body(i, carry):
            base = pl.multiple_of(ebase + i * CHUNK, 8)
            pltpu.sync_copy(dst_hbm.at[pl.ds(base, CHUNK)], dst_v)
            pltpu.sync_copy(ones_v, acc_sp.at[dst_v], add=True)
            return carry

        lax.fori_loop(0, NCHUNKS, body, 0)
        plsc.subcore_barrier()
        pltpu.sync_copy(acc_sp.at[pl.ds(r0, RPT)],
                        out_hbm.at[cid, pl.ds(r0, RPT)])

    return k(dst)


def _sc_scatter_1d(vals, src, dst):
    @functools.partial(
        pl.kernel,
        out_type=jax.ShapeDtypeStruct((NC, N_PAD), jnp.float32),
        mesh=_MESH,
        scratch_types=[
            pltpu.VMEM((CHUNK,), jnp.int32),
            pltpu.VMEM((CHUNK,), jnp.int32),
            pltpu.VMEM((CHUNK,), jnp.float32),
            pltpu.VMEM((RPT,), jnp.float32),
            pltpu.VMEM_SHARED((N_PAD,), jnp.float32),
            pltpu.SemaphoreType.DMA,
        ],
    )
    def k(v_hbm, src_hbm, dst_hbm, out_hbm, src_v, dst_v, vals_v, zbuf,
          acc_sp, sem):
        cid = lax.axis_index("c")
        sid = lax.axis_index("s")
        for j in range(RPT // 16):
            zbuf[pl.ds(j * 16, 16)] = jnp.zeros((16,), jnp.float32)
        r0 = sid * RPT
        pltpu.sync_copy(zbuf, acc_sp.at[pl.ds(r0, RPT)])
        plsc.subcore_barrier()

        ebase = (cid * NS + sid) * E_PER_TILE

        def body(i, carry):
            base = pl.multiple_of(ebase + i * CHUNK, 8)
            pltpu.sync_copy(src_hbm.at[pl.ds(base, CHUNK)], src_v)
            pltpu.sync_copy(dst_hbm.at[pl.ds(base, CHUNK)], dst_v)
            pltpu.async_copy(v_hbm.at[src_v], vals_v, sem).wait()
            pltpu.sync_copy(vals_v, acc_sp.at[dst_v], add=True)
            return carry

        lax.fori_loop(0, NCHUNKS, body, 0)
        plsc.subcore_barrier()
        pltpu.sync_copy(acc_sp.at[pl.ds(r0, RPT)],
                        out_hbm.at[cid, pl.ds(r0, RPT)])

    return k(vals, src, dst)


def _tc_first(x, W0, degp):
    def body(x_ref, w_ref, degp_ref, d_ref, h_ref):
        deg = (degp_ref[0, :N] + degp_ref[1, :N] + 1.0).reshape(N, 1)
        d = lax.rsqrt(deg)
        d_ref[...] = d
        h = jnp.dot(x_ref[...], w_ref[...],
                    preferred_element_type=jnp.float32)
        h_ref[...] = h * d

    return pl.pallas_call(
        body,
        out_shape=(jax.ShapeDtypeStruct((N, 1), jnp.float32),
                   jax.ShapeDtypeStruct((N, D), jnp.float32)),
    )(x, W0, degp)


def _tc_mid(acc, h, d, b, g, be, Wn, d_out):
    def body(acc_ref, h_ref, d_ref, b_ref, g_ref, be_ref, w_ref, o_ref):
        d_ = d_ref[...]
        z = d_ * (acc_ref[0, :N, :] + acc_ref[1, :N, :] + h_ref[...]) \
            + b_ref[...]
        mean = jnp.mean(z, axis=0, keepdims=True)
        zc = z - mean
        var = jnp.mean(zc * zc, axis=0, keepdims=True)
        y = g_ref[...] * zc * lax.rsqrt(var + EPS) + be_ref[...]
        y = jnp.maximum(y, 0.0)
        o_ref[...] = jnp.dot(y, w_ref[...],
                             preferred_element_type=jnp.float32) * d_

    return pl.pallas_call(
        body,
        out_shape=jax.ShapeDtypeStruct((N, d_out), jnp.float32),
    )(acc, h, d, b.reshape(1, D), g.reshape(1, D), be.reshape(1, D), Wn)


def _tc_final(acc, h3, d, b3):
    def body(acc_ref, h3_ref, d_ref, b3_ref, o_ref):
        a = (acc_ref[0, :N] + acc_ref[1, :N]).reshape(N, 1)
        o_ref[...] = d_ref[...] * (a + h3_ref[...]) + b3_ref[...]

    return pl.pallas_call(
        body,
        out_shape=jax.ShapeDtypeStruct((N, 1), jnp.float32),
    )(acc, h3, d, b3.reshape(1, 1))


def kernel(x, edge_index, W0, b0, W1, b1, W2, b2, W3, b3,
           g0, be0, g1, be1, g2, be2):
    src = edge_index[0]
    dst = edge_index[1]
    zblk = jnp.zeros((RPT, D), jnp.float32)

    degp = _sc_deg(dst)
    d, h = _tc_first(x, W0, degp)

    acc = _sc_scatter_wide(h, src, dst, zblk)
    h = _tc_mid(acc, h, d, b0, g0, be0, W1, D)

    acc = _sc_scatter_wide(h, src, dst, zblk)
    h = _tc_mid(acc, h, d, b1, g1, be1, W2, D)

    acc = _sc_scatter_wide(h, src, dst, zblk)
    h3 = _tc_mid(acc, h, d, b2, g2, be2, W3, 1)

    acc3 = _sc_scatter_1d(h3.reshape(-1), src, dst)
    out = _tc_final(acc3, h3, d, b3)
    return out.reshape(-1)
